# baseline (speedup 1.0000x reference)
import functools

import jax
import jax.numpy as jnp
import numpy as np
from jax import lax
from jax.experimental import pallas as pl
from jax.experimental.pallas import tpu as pltpu
from jax.experimental.pallas import tpu_sc as plsc

D = 1024
H = 16
DH = D // H
NTOK = 676
SEL = int(676 * 0.8)
NLOW = NTOK - SEL
B = 8
L = 77
NF = NTOK * B
TOKP = 544
NA = B * TOKP
NTRASH = 8
NAB = NA + NTRASH
NW = 32
SSTR = 168
SREAD = 200
SPAD = 208
NG = 5632
RGW = NG // NW


def _score_body(body_ref, lang_ref, out_ref):
    cols = []
    for b in range(B):
        tb = body_ref[:, b, :].astype(jnp.bfloat16)
        lb = lang_ref[:, b, :].astype(jnp.bfloat16)
        p = lax.dot_general(tb, lb, (((1,), (1,)), ((), ())),
                            preferred_element_type=jnp.float32)
        cols.append(jnp.sum(p, axis=1, keepdims=True) * (1.0 / L))
    out_ref[:] = jnp.concatenate(cols, axis=1)


def _select_body(s_nb_ref, s_bn_ref, dst_ref, src_ref, w_ref):
    s_nb = s_nb_ref[:]
    s_bn = s_bn_ref[:]
    f32 = jnp.float32

    row_i = lax.broadcasted_iota(jnp.int32, (NTOK, NTOK), 0)
    col_i = lax.broadcasted_iota(jnp.int32, (NTOK, NTOK), 1)
    lt_mn = (row_i < col_i)
    lt_f = lt_mn.astype(f32)

    ranks = []
    for b in range(B):
        cm = s_nb[:, b:b + 1]
        cn = s_bn[b:b + 1, :]
        gt = (cm > cn).astype(f32)
        tie = jnp.where((cm == cn) & lt_mn, 1.0, 0.0)
        ranks.append(jnp.sum(gt + tie, axis=0, keepdims=True))
    rank_bn = jnp.concatenate(ranks, axis=0)
    keep = rank_bn < float(SEL)
    kf = keep.astype(f32)
    df = 1.0 - kf

    bi = lax.broadcasted_iota(jnp.int32, (B, B), 0)
    bj = lax.broadcasted_iota(jnp.int32, (B, B), 1)
    u_bb = (bj < bi).astype(f32)

    def excl_prefix(flag_bn):
        cnt = jnp.sum(flag_bn, axis=0, keepdims=True)
        pref = lax.dot_general(cnt, lt_f, (((1,), (0,)), ((), ())),
                               preferred_element_type=f32,
                               precision=lax.Precision.HIGHEST)
        intra = lax.dot_general(u_bb, flag_bn, (((1,), (0,)), ((), ())),
                                preferred_element_type=f32,
                               precision=lax.Precision.HIGHEST)
        return pref + intra

    d_hi = excl_prefix(kf).astype(jnp.int32)
    d_lo_f = excl_prefix(df)
    d_lo = d_lo_f.astype(jnp.int32)

    hi_row = (d_hi & 7) * TOKP + 1 + (d_hi >> 3)
    jlo = jnp.zeros_like(d_lo_f)
    for t in range(1, B):
        jlo = jlo + (d_lo_f >= float(NLOW * t)).astype(f32)
    jlo_i = jlo.astype(jnp.int32)

    dst_ref[:] = jnp.where(keep, hi_row, NA + (d_lo & 7))
    src_ref[:] = jnp.where(keep, hi_row, jlo_i * TOKP + (TOKP - 3))

    neg = jnp.float32(-3e38)
    for j in range(B):
        m = (~keep) & (jlo_i == j)
        sm = jnp.where(m, s_bn, neg)
        mx = jnp.max(jnp.max(sm, axis=1, keepdims=True), axis=0, keepdims=True)
        e = jnp.where(m, jnp.exp(s_bn - mx), 0.0)
        z = jnp.sum(jnp.sum(e, axis=1, keepdims=True), axis=0, keepdims=True)
        w_ref[j, :, :] = e / z


def _lowx_body(w_ref, body_ref, out_ref):
    @pl.when(pl.program_id(0) == 0)
    def _():
        out_ref[:] = jnp.zeros_like(out_ref)
    out_ref[:] += lax.dot_general(w_ref[:], body_ref[:],
                                  (((0,), (0,)), ((), ())),
                                  preferred_element_type=jnp.float32,
                        precision=lax.Precision.HIGHEST)


def _lnqkv_body(a_ref, g_ref, b_ref, w_ref, bias_ref, o_ref):
    t = a_ref[:]
    m = jnp.mean(t, axis=-1, keepdims=True)
    v = jnp.mean((t - m) ** 2, axis=-1, keepdims=True)
    h = (t - m) * lax.rsqrt(v + 1e-5) * g_ref[:] + b_ref[:]
    o_ref[:] = lax.dot_general(h, w_ref[:], (((1,), (1,)), ((), ())),
                               preferred_element_type=jnp.float32,
                        precision=lax.Precision.HIGHEST) + bias_ref[:]


def _attn_body(q_ref, k_ref, v_ref, o_ref):
    col = lax.broadcasted_iota(jnp.int32, (TOKP, TOKP), 1)
    rowv = lax.broadcasted_iota(jnp.int32, (TOKP, DH), 0)
    outs = []
    for h in range(2):
        q = q_ref[:, h * DH:(h + 1) * DH]
        k = k_ref[:, h * DH:(h + 1) * DH]
        v = v_ref[:, h * DH:(h + 1) * DH]
        s = lax.dot_general(q, k, (((1,), (1,)), ((), ())),
                            preferred_element_type=jnp.float32,
                        precision=lax.Precision.HIGHEST) * (1.0 / 8.0)
        s = jnp.where(col >= TOKP - 2, jnp.float32(-1e30), s)
        mx = jnp.max(s, axis=1, keepdims=True)
        e = jnp.exp(s - mx)
        p = e / jnp.sum(e, axis=1, keepdims=True)
        v = jnp.where(rowv >= TOKP - 2, 0.0, v)
        outs.append(lax.dot_general(p, v, (((1,), (0,)), ((), ())),
                                    preferred_element_type=jnp.float32,
                        precision=lax.Precision.HIGHEST))
    o_ref[:] = jnp.concatenate(outs, axis=1)


def _proj_body(a_ref, w_ref, b_ref, o_ref):
    o_ref[:] = lax.dot_general(a_ref[:], w_ref[:], (((1,), (1,)), ((), ())),
                               preferred_element_type=jnp.float32,
                        precision=lax.Precision.HIGHEST) + b_ref[:]


def _mlp_body(x_ref, gth_ref, g_ref, b_ref, w1_ref, b1_ref, w2_ref, b2_ref,
              o_ref):
    u = x_ref[:] + gth_ref[:]
    m = jnp.mean(u, axis=-1, keepdims=True)
    v = jnp.mean((u - m) ** 2, axis=-1, keepdims=True)
    h = (u - m) * lax.rsqrt(v + 1e-5) * g_ref[:] + b_ref[:]
    a = lax.dot_general(h, w1_ref[:], (((1,), (1,)), ((), ())),
                        preferred_element_type=jnp.float32,
                        precision=lax.Precision.HIGHEST) + b1_ref[:]
    a = a * (1.0 / (1.0 + jnp.exp(-1.702 * a)))
    y = lax.dot_general(a, w2_ref[:], (((1,), (1,)), ((), ())),
                        preferred_element_type=jnp.float32,
                        precision=lax.Precision.HIGHEST) + b2_ref[:]
    o_ref[:] = u + y


@functools.lru_cache(maxsize=1)
def _sc_kernels():
    mesh = plsc.VectorSubcoreMesh(core_axis_name="c", subcore_axis_name="s")

    @functools.partial(
        pl.kernel,
        out_type=jax.ShapeDtypeStruct((NAB, D), jnp.float32),
        mesh=mesh,
        scratch_types=[
            pltpu.VMEM((2, 104), jnp.int32),
            pltpu.VMEM((104, D), jnp.float32),
            pltpu.VMEM((B, D), jnp.float32),
            pltpu.VMEM((1, B), jnp.int32),
            pltpu.SemaphoreType.DMA,
        ],
    )
    def scatter_build(body_hbm, x0_hbm, lowx_hbm, dst_hbm, clsidx_hbm,
                      lowidx_hbm, a2_hbm, idx_v, data_v, row8_v, idx8_v, sem):
        wid = lax.axis_index("s") * 2 + lax.axis_index("c")
        pltpu.sync_copy(dst_hbm.at[wid], idx_v)
        base = wid * SSTR
        for ci, (off, nsz) in enumerate(((0, 104), (104, SREAD - 104))):
            pltpu.sync_copy(body_hbm.at[pl.ds(base + off, nsz)],
                            data_v.at[pl.ds(0, nsz)])
            pltpu.async_copy(data_v, a2_hbm.at[idx_v.at[ci]], sem).wait()

        @pl.when(wid == 0)
        def _():
            pltpu.sync_copy(x0_hbm, row8_v)
            pltpu.sync_copy(clsidx_hbm, idx8_v)
            pltpu.async_copy(row8_v, a2_hbm.at[idx8_v.at[0]], sem).wait()
            pltpu.sync_copy(lowx_hbm, row8_v)
            pltpu.sync_copy(lowidx_hbm, idx8_v)
            pltpu.async_copy(row8_v, a2_hbm.at[idx8_v.at[0]], sem).wait()

    @functools.partial(
        pl.kernel,
        out_type=jax.ShapeDtypeStruct((NG, D), jnp.float32),
        mesh=mesh,
        scratch_types=[
            pltpu.VMEM((RGW,), jnp.int32),
            pltpu.VMEM((88, D), jnp.float32),
            pltpu.SemaphoreType.DMA,
        ],
    )
    def gather_merge(delta_hbm, src_hbm, g_hbm, idx_v, data_v, sem):
        wid = lax.axis_index("s") * 2 + lax.axis_index("c")
        pltpu.sync_copy(src_hbm.at[wid], idx_v)
        for off, nsz in ((0, 88), (88, 88)):
            pltpu.async_copy(delta_hbm.at[idx_v.at[pl.ds(off, nsz)]],
                             data_v.at[pl.ds(0, nsz)], sem).wait()
            pltpu.sync_copy(data_v.at[pl.ds(0, nsz)],
                            g_hbm.at[pl.ds(wid * RGW + off, nsz)])

    return scatter_build, gather_merge


def _sc_scatter_build(body_flat, x0, lowx, dst_pad, cls_idx, low_idx):
    return _sc_kernels()[0](body_flat, x0, lowx, dst_pad, cls_idx, low_idx)


def _sc_gather_merge(delta, src_pad):
    return _sc_kernels()[1](delta, src_pad)


def kernel(x, lang_tokens, w_in, b_in, w_out, b_out, ln1_g, ln1_b, ln2_g,
           ln2_b, fc1_w, fc1_b, fc2_w, fc2_b):
    f32 = jnp.float32
    body = x[1:]
    body_flat = body.reshape(NF, D)
    x0 = x[0]

    score = pl.pallas_call(
        _score_body,
        grid=(1,),
        in_specs=[pl.BlockSpec((NTOK, B, D), lambda i: (0, 0, 0)),
                  pl.BlockSpec((L, B, D), lambda i: (0, 0, 0))],
        out_specs=pl.BlockSpec((NTOK, B), lambda i: (0, 0)),
        out_shape=jax.ShapeDtypeStruct((NTOK, B), f32),
    )(body, lang_tokens)

    dst_bn, src_bn, w_jbn = pl.pallas_call(
        _select_body,
        grid=(1,),
        in_specs=[pl.BlockSpec((NTOK, B), lambda i: (0, 0)),
                  pl.BlockSpec((B, NTOK), lambda i: (0, 0))],
        out_specs=[pl.BlockSpec((B, NTOK), lambda i: (0, 0)),
                   pl.BlockSpec((B, NTOK), lambda i: (0, 0)),
                   pl.BlockSpec((B, B, NTOK), lambda i: (0, 0, 0))],
        out_shape=[jax.ShapeDtypeStruct((B, NTOK), jnp.int32),
                   jax.ShapeDtypeStruct((B, NTOK), jnp.int32),
                   jax.ShapeDtypeStruct((B, B, NTOK), f32)],
    )(score, score.T)

    dst_flat = dst_bn.T.reshape(NF)
    widx = (jnp.arange(NW, dtype=jnp.int32)[:, None] * SSTR
            + jnp.arange(SPAD, dtype=jnp.int32)[None, :])
    dst_pad = jnp.where(jnp.arange(SPAD)[None, :] < SREAD,
                        dst_flat[jnp.minimum(widx, NF - 1)],
                        jnp.int32(NA)).reshape(NW, 2, 104)
    src_flat = jnp.concatenate(
        [jnp.arange(B, dtype=jnp.int32) * TOKP, src_bn.T.reshape(NF),
         jnp.zeros((NG - B - NF,), jnp.int32)])
    src_pad = src_flat.reshape(NW, RGW)
    w_mat = w_jbn.transpose(0, 2, 1).reshape(B, NF).T
    cls_idx = (jnp.arange(B, dtype=jnp.int32) * TOKP).reshape(1, B)
    low_idx = cls_idx + (TOKP - 3)

    lowx = pl.pallas_call(
        _lowx_body,
        grid=(4,),
        in_specs=[pl.BlockSpec((NF // 4, B), lambda i: (i, 0)),
                  pl.BlockSpec((NF // 4, D), lambda i: (i, 0))],
        out_specs=pl.BlockSpec((B, D), lambda i: (0, 0)),
        out_shape=jax.ShapeDtypeStruct((B, D), f32),
    )(w_mat, body_flat)

    a2 = _sc_scatter_build(body_flat, x0, lowx, dst_pad, cls_idx, low_idx)

    qkv = pl.pallas_call(
        _lnqkv_body,
        grid=(B,),
        in_specs=[pl.BlockSpec((TOKP, D), lambda i: (i, 0)),
                  pl.BlockSpec((1, D), lambda i: (0, 0)),
                  pl.BlockSpec((1, D), lambda i: (0, 0)),
                  pl.BlockSpec((3 * D, D), lambda i: (0, 0)),
                  pl.BlockSpec((1, 3 * D), lambda i: (0, 0))],
        out_specs=pl.BlockSpec((TOKP, 3 * D), lambda i: (i, 0)),
        out_shape=jax.ShapeDtypeStruct((NA, 3 * D), f32),
    )(a2, ln1_g.reshape(1, D), ln1_b.reshape(1, D), w_in,
      b_in.reshape(1, 3 * D))

    attn = pl.pallas_call(
        _attn_body,
        grid=(B, H // 2),
        in_specs=[pl.BlockSpec((TOKP, 2 * DH), lambda b, h: (b, h)),
                  pl.BlockSpec((TOKP, 2 * DH), lambda b, h: (b, H // 2 + h)),
                  pl.BlockSpec((TOKP, 2 * DH), lambda b, h: (b, H + h))],
        out_specs=pl.BlockSpec((TOKP, 2 * DH), lambda b, h: (b, h)),
        out_shape=jax.ShapeDtypeStruct((NA, D), f32),
    )(qkv, qkv, qkv)

    delta = pl.pallas_call(
        _proj_body,
        grid=(B,),
        in_specs=[pl.BlockSpec((TOKP, D), lambda i: (i, 0)),
                  pl.BlockSpec((D, D), lambda i: (0, 0)),
                  pl.BlockSpec((1, D), lambda i: (0, 0))],
        out_specs=pl.BlockSpec((TOKP, D), lambda i: (i, 0)),
        out_shape=jax.ShapeDtypeStruct((NA, D), f32),
    )(attn, w_out, b_out.reshape(1, D))

    gth = _sc_gather_merge(delta, src_pad)

    x_flat = x.reshape((NTOK + 1) * B, D)
    nrows = (NTOK + 1) * B
    blk = 248
    out = pl.pallas_call(
        _mlp_body,
        grid=(22,),
        in_specs=[pl.BlockSpec((blk, D), lambda i: (i, 0)),
                  pl.BlockSpec((blk, D), lambda i: (i, 0)),
                  pl.BlockSpec((1, D), lambda i: (0, 0)),
                  pl.BlockSpec((1, D), lambda i: (0, 0)),
                  pl.BlockSpec((4 * D, D), lambda i: (0, 0)),
                  pl.BlockSpec((1, 4 * D), lambda i: (0, 0)),
                  pl.BlockSpec((D, 4 * D), lambda i: (0, 0)),
                  pl.BlockSpec((1, D), lambda i: (0, 0))],
        out_specs=pl.BlockSpec((blk, D), lambda i: (i, 0)),
        out_shape=jax.ShapeDtypeStruct((nrows, D), f32),
    )(x_flat, gth, ln2_g.reshape(1, D), ln2_b.reshape(1, D), fc1_w,
      fc1_b.reshape(1, 4 * D), fc2_w, fc2_b.reshape(1, D))

    return out.reshape(NTOK + 1, B, D)
